# baseline (speedup 1.0000x reference)
import jax
import jax.numpy as jnp
from jax import lax
from jax.experimental import pallas as pl
from jax.experimental.pallas import tpu as pltpu
from jax.experimental.pallas import tpu_sc as plsc


_W = 128
_BATCH = 16384
_FIELDS = 26
_EMB = 32
_UNITS = _FIELDS * (_BATCH // _W)
_NW = 32
_UPW = _UNITS // _NW


def _gather_tiled(table_p, idx_flat):
    mesh = plsc.VectorSubcoreMesh(core_axis_name="core",
                                  subcore_axis_name="subcore")

    @pl.kernel(
        out_type=jax.ShapeDtypeStruct((_FIELDS, _EMB, _BATCH), jnp.float32),
        mesh=mesh,
        compiler_params=pltpu.CompilerParams(use_tc_tiling_on_sc=True,
                                             needs_layout_passes=False),
        scratch_types=[
            pltpu.VMEM((_W,), jnp.int32),
            pltpu.VMEM((_W,), jnp.int32),
            pltpu.VMEM((_W,), jnp.int32),
            pltpu.VMEM((_W, 128), jnp.float32),
            pltpu.VMEM((_EMB, _W), jnp.float32),
            pltpu.SemaphoreType.DMA,
        ],
    )
    def kern(tab_hbm, idx_hbm, out_hbm, idx_v, hi_v, colb_v, rows_v, outb_v,
             sem):
        wid = lax.axis_index("subcore") * 2 + lax.axis_index("core")
        u0 = wid * _UPW

        @pl.loop(0, _UPW)
        def _(k):
            u = u0 + k
            f = u >> 7
            c = u & 127
            pltpu.sync_copy(idx_hbm.at[pl.ds(u * _W, _W)], idx_v)
            for w0 in range(_W // 16):
                v = idx_v[pl.ds(w0 * 16, 16)]
                hi_v[pl.ds(w0 * 16, 16)] = lax.shift_right_logical(v, 2)
                colb_v[pl.ds(w0 * 16, 16)] = (v & 3) * _EMB
            pltpu.async_copy(tab_hbm.at[hi_v], rows_v, sem).wait()
            for w0 in range(_W // 16):
                row_i = w0 * 16 + lax.iota(jnp.int32, 16)
                colb = colb_v[pl.ds(w0 * 16, 16)]
                for e in range(_EMB):
                    g = plsc.load_gather(rows_v, [row_i, colb + e])
                    outb_v.at[e][pl.ds(w0 * 16, 16)] = g
            pltpu.sync_copy(outb_v, out_hbm.at[f, :, pl.ds(c * _W, _W)])

    return kern(table_p, idx_flat)


def kernel(inputs, embedding_weights):
    table_p = jnp.reshape(embedding_weights, (250000, 128))
    idx_flat = jnp.reshape(jnp.transpose(inputs), (-1,)).astype(jnp.int32)
    out_fm = _gather_tiled(table_p, idx_flat)
    return jnp.transpose(out_fm, (2, 0, 1))


# speedup vs baseline: 1.1411x; 1.1411x over previous
import jax
import jax.numpy as jnp
from jax import lax
from jax.experimental import pallas as pl
from jax.experimental.pallas import tpu as pltpu
from jax.experimental.pallas import tpu_sc as plsc


_W = 128
_BATCH = 16384
_FIELDS = 26
_EMB = 32
_UNITS = _FIELDS * (_BATCH // _W)
_NW = 32
_UPW = _UNITS // _NW


def _gather_native(table_rows, idx_flat):
    mesh = plsc.VectorSubcoreMesh(core_axis_name="core",
                                  subcore_axis_name="subcore")

    @pl.kernel(
        out_type=jax.ShapeDtypeStruct((_FIELDS, 4, _BATCH // _W, 8 * _W),
                                      jnp.float32),
        mesh=mesh,
        compiler_params=pltpu.CompilerParams(use_tc_tiling_on_sc=False,
                                             needs_layout_passes=False),
        scratch_types=[
            pltpu.VMEM((_W,), jnp.int32),
            pltpu.VMEM((_W, _EMB), jnp.float32),
            pltpu.VMEM((4, 8 * _W), jnp.float32),
            pltpu.SemaphoreType.DMA,
        ],
    )
    def kern(tab_hbm, idx_hbm, out_hbm, idx_v, rows_v, outb_v, sem):
        wid = lax.axis_index("subcore") * 2 + lax.axis_index("core")
        u0 = wid * _UPW
        iota = lax.iota(jnp.int32, 16)

        @pl.loop(0, _UPW)
        def _(k):
            u = u0 + k
            f = u >> 7
            c = u & 127
            pltpu.sync_copy(idx_hbm.at[pl.ds(u * _W, _W)], idx_v)
            pltpu.async_copy(tab_hbm.at[idx_v], rows_v, sem).wait()
            rlo = iota // 8
            rhi = rlo + 2
            clo = (iota % 8) * _W
            chi = clo
            for w in range(_W):
                wv = jnp.full((16,), w, jnp.int32)
                a = plsc.load_gather(rows_v, [wv, iota])
                b = plsc.load_gather(rows_v, [wv, iota + 16])
                plsc.store_scatter(outb_v, [rlo, clo + w], a)
                plsc.store_scatter(outb_v, [rhi, chi + w], b)
            pltpu.sync_copy(outb_v, out_hbm.at[f, :, c, :])

    return kern(table_rows, idx_flat)


def kernel(inputs, embedding_weights):
    table_p = jnp.reshape(embedding_weights, (250000, 128))
    table_rows = jnp.reshape(table_p, (1000000, 32))
    idx_flat = jnp.reshape(jnp.transpose(inputs), (-1,)).astype(jnp.int32)
    r = _gather_native(table_rows, idx_flat)
    r5 = jnp.reshape(r, (_FIELDS, 4, _BATCH // _W, 8, _W))
    out = jnp.transpose(r5, (2, 4, 0, 1, 3))
    return jnp.reshape(out, (_BATCH, _FIELDS, _EMB))


# speedup vs baseline: 1.2070x; 1.0577x over previous
import jax
import jax.numpy as jnp
from jax import lax
from jax.experimental import pallas as pl
from jax.experimental.pallas import tpu as pltpu
from jax.experimental.pallas import tpu_sc as plsc


_W = 512
_BATCH = 16384
_FIELDS = 26
_EMB = 32
_UNITS = _FIELDS * (_BATCH // _W)
_NW = 32
_UPW = _UNITS // _NW
_WIN = _BATCH // _W


def _gather_native(table_rows, idx_flat):
    mesh = plsc.VectorSubcoreMesh(core_axis_name="core",
                                  subcore_axis_name="subcore")

    @pl.kernel(
        out_type=jax.ShapeDtypeStruct((_FIELDS, 4, _WIN, 8 * _W),
                                      jnp.float32),
        mesh=mesh,
        compiler_params=pltpu.CompilerParams(use_tc_tiling_on_sc=False,
                                             needs_layout_passes=False),
        scratch_types=[
            pltpu.VMEM((_W,), jnp.int32),
            pltpu.VMEM((_W,), jnp.int32),
            pltpu.VMEM((_W, _EMB), jnp.float32),
            pltpu.VMEM((_W, _EMB), jnp.float32),
            pltpu.VMEM((4, 8 * _W), jnp.float32),
            pltpu.SemaphoreType.DMA,
            pltpu.SemaphoreType.DMA,
        ],
    )
    def kern(tab_hbm, idx_hbm, out_hbm, idx0, idx1, rows0, rows1, outb_v,
             sem0, sem1):
        wid = lax.axis_index("subcore") * 2 + lax.axis_index("core")
        u0 = wid * _UPW
        iota = lax.iota(jnp.int32, 16)

        def fetch(u, idx_b, rows_b, sem):
            pltpu.sync_copy(idx_hbm.at[pl.ds(u * _W, _W)], idx_b)
            return pltpu.async_copy(tab_hbm.at[idx_b], rows_b, sem)

        def flush(u, rows_b):
            @pl.loop(0, _W // 16)
            def _(j):
                w0 = j * 16
                rowsel = w0 + iota
                cbase = (j >> 3) * 1024 + (j & 7) * 16
                for e in range(_EMB):
                    v = plsc.load_gather(rows_b,
                                         [rowsel, jnp.full((16,), e, jnp.int32)])
                    outb_v.at[e >> 3][pl.ds(cbase + (e & 7) * 128, 16)] = v
            f = u // _WIN
            c = lax.rem(u, _WIN)
            pltpu.sync_copy(outb_v, out_hbm.at[f, :, c, :])

        cp0 = fetch(u0, idx0, rows0, sem0)
        del cp0

        @pl.loop(0, _UPW, step=2)
        def _(k):
            u = u0 + k

            @pl.when(k + 1 < _UPW)
            def _():
                fetch(u + 1, idx1, rows1, sem1)

            pltpu.make_async_copy(tab_hbm.at[idx0], rows0, sem0).wait()
            flush(u, rows0)

            @pl.when(k + 2 < _UPW)
            def _():
                fetch(u + 2, idx0, rows0, sem0)

            @pl.when(k + 1 < _UPW)
            def _():
                pltpu.make_async_copy(tab_hbm.at[idx1], rows1, sem1).wait()
                flush(u + 1, rows1)

    return kern(table_rows, idx_flat)


def kernel(inputs, embedding_weights):
    table_rows = jnp.reshape(jnp.reshape(embedding_weights, (250000, 128)),
                             (1000000, 32))
    idx_flat = jnp.reshape(jnp.transpose(inputs), (-1,)).astype(jnp.int32)
    r = _gather_native(table_rows, idx_flat)
    r5 = jnp.reshape(r, (_FIELDS, 4, 128, 8, 128))
    out = jnp.transpose(r5, (2, 4, 0, 1, 3))
    return jnp.reshape(out, (_BATCH, _FIELDS, _EMB))


# speedup vs baseline: 1.3065x; 1.0824x over previous
import jax
import jax.numpy as jnp
from jax.experimental import pallas as pl
from jax.experimental.pallas import tpu as pltpu
from jax.experimental.pallas import tpu_sc as plsc


_WINDOW = 1024


def _gather_rows(table, idx_fm, batch, fields):
    emb = table.shape[1]
    mesh = plsc.VectorSubcoreMesh(core_axis_name="core",
                                  subcore_axis_name="subcore")

    @pl.kernel(out_type=jax.ShapeDtypeStruct((batch, fields, emb), table.dtype),
               mesh=mesh,
               compiler_params=pltpu.CompilerParams(use_tc_tiling_on_sc=False))
    def kern(x_hbm, i_hbm, o_hbm):
        def body(i_vmem, o_vmem):
            pltpu.sync_copy(x_hbm.at[i_vmem.at[0]], o_vmem.at[:, 0])

        pltpu.emit_pipeline(
            body,
            grid=(fields, batch // _WINDOW),
            in_specs=[pl.BlockSpec((1, _WINDOW), index_map=lambda f, j: (f, j))],
            out_specs=[pl.BlockSpec((_WINDOW, 1, emb),
                                    index_map=lambda f, j: (j, f, 0))],
            core_axis_name=("core", "subcore"),
            dimension_semantics=(pltpu.PARALLEL, pltpu.PARALLEL),
        )(i_hbm, o_hbm)

    return kern(table, idx_fm)


def kernel(inputs, embedding_weights):
    batch, fields = inputs.shape
    idx_fm = jnp.transpose(inputs).astype(jnp.int32)
    return _gather_rows(embedding_weights, idx_fm, batch, fields)
